# baseline (speedup 1.0000x reference)
import functools

import jax
import jax.numpy as jnp
from jax import lax
from jax.experimental import pallas as pl
from jax.experimental.pallas import tpu as pltpu
from jax.experimental.pallas import tpu_sc as plsc

N = 50000
E = 800000
G = 512
H = 64

NC = 2
NS = 16
NW = NC * NS
CH = 128
CPW = 200
EPAD = NW * CPW * CH
NCHUNK = EPAD // CH
NPAD = 50048
RPT = NPAD // NS
GPAD = 520
NB = 3128
NBLK = NPAD // NB
NDEPTH = 4
GRP = 40
PRE = 2

_f32 = jnp.float32


def _mesh():
    return plsc.VectorSubcoreMesh(core_axis_name="c", subcore_axis_name="s")


def _edge_phase(val_hbm, srcp, dstp, sidx, didx, rbufs, sems, acc,
                base0, ngroups):
    gsems, ssems = sems
    ndepth = len(rbufs)
    pre = ndepth // 2

    @pl.loop(0, ngroups)
    def _groups(g):
        base = base0 + g * GRP
        pltpu.sync_copy(srcp.at[pl.ds(base, GRP)], sidx)
        pltpu.sync_copy(dstp.at[pl.ds(base, GRP)], didx)
        for b in range(pre):
            pltpu.async_copy(val_hbm.at[sidx.at[b]], rbufs[b], gsems[b])

        @pl.loop(0, GRP, step=ndepth)
        def _chunks(k):
            for b in range(ndepth):
                j = k + b
                pltpu.make_async_copy(
                    val_hbm.at[pl.ds(0, CH)], rbufs[b], gsems[b]
                ).wait()
                pltpu.async_copy(rbufs[b], acc.at[didx.at[j]], ssems[b],
                                 add=True)
                b2 = (b + pre) % ndepth

                @pl.when(j >= pre)
                def _():
                    pltpu.make_async_copy(
                        rbufs[b2], acc.at[pl.ds(0, CH)], ssems[b2]
                    ).wait()

                @pl.when(j + pre < GRP)
                def _():
                    pltpu.async_copy(val_hbm.at[sidx.at[j + pre]],
                                     rbufs[b2], gsems[b2])

        for j in range(GRP - pre, GRP):
            b = j % ndepth
            pltpu.make_async_copy(
                rbufs[b], acc.at[pl.ds(0, CH)], ssems[b]
            ).wait()


def _agg16_body(xe, srcp, dstp, z16, p0, p1,
                acc, sidx, didx, rb0, rb1, rb2, rb3, rb4, rb5, rb6, rb7,
                gs0, gs1, gs2, gs3, gs4, gs5, gs6, gs7,
                ss0, ss1, ss2, ss3, ss4, ss5, ss6, ss7):
    c = lax.axis_index("c")
    s = lax.axis_index("s")
    w = s * NC + c
    r0 = s * RPT
    pltpu.sync_copy(z16.at[pl.ds(r0, RPT)], acc.at[pl.ds(r0, RPT)])
    plsc.subcore_barrier()

    _edge_phase(xe, srcp, dstp, sidx, didx,
                (rb0, rb1, rb2, rb3, rb4, rb5, rb6, rb7),
                ((gs0, gs1, gs2, gs3, gs4, gs5, gs6, gs7),
                 (ss0, ss1, ss2, ss3, ss4, ss5, ss6, ss7)),
                acc, w * CPW, CPW // GRP)
    plsc.subcore_barrier()

    @pl.when(c == 0)
    def _():
        pltpu.sync_copy(acc.at[pl.ds(r0, RPT)], p0.at[pl.ds(r0, RPT)])

    @pl.when(c == 1)
    def _():
        pltpu.sync_copy(acc.at[pl.ds(r0, RPT)], p1.at[pl.ds(r0, RPT)])


def _agg32_body(ha, hb, srcp, dstp, z32, sa, sb,
                acc, sidx, didx, rb0, rb1, rb2, rb3,
                gs0, gs1, gs2, gs3, ss0, ss1, ss2, ss3):
    c = lax.axis_index("c")
    s = lax.axis_index("s")
    r0 = s * RPT
    pltpu.sync_copy(z32.at[pl.ds(r0, RPT)], acc.at[pl.ds(r0, RPT)])
    plsc.subcore_barrier()

    rbufs = (rb0, rb1, rb2, rb3)
    sems = ((gs0, gs1, gs2, gs3), (ss0, ss1, ss2, ss3))
    base0 = s * (2 * CPW)
    ngroups = 2 * CPW // GRP

    @pl.when(c == 0)
    def _():
        _edge_phase(ha, srcp, dstp, sidx, didx, rbufs, sems, acc,
                    base0, ngroups)

    @pl.when(c == 1)
    def _():
        _edge_phase(hb, srcp, dstp, sidx, didx, rbufs, sems, acc,
                    base0, ngroups)

    plsc.subcore_barrier()

    @pl.when(c == 0)
    def _():
        pltpu.sync_copy(acc.at[pl.ds(r0, RPT)], sa.at[pl.ds(r0, RPT)])

    @pl.when(c == 1)
    def _():
        pltpu.sync_copy(acc.at[pl.ds(r0, RPT)], sb.at[pl.ds(r0, RPT)])


def _pool_body(h3, batchp, ones16, zg64, zg16, ps0, pc0, ps1, pc1,
               acc, cacc, bidx, vb, onesv):
    c = lax.axis_index("c")
    s = lax.axis_index("s")
    w = s * NC + c

    @pl.when(s == 0)
    def _():
        pltpu.sync_copy(zg64, acc)
        pltpu.sync_copy(zg16, cacc)

    pltpu.sync_copy(ones16, onesv)
    plsc.subcore_barrier()

    nrows = NPAD // CH
    nj = (nrows - w + NW - 1) // NW

    @pl.loop(0, nj)
    def _rows(k):
        j = w + k * NW
        pltpu.sync_copy(batchp.at[j], bidx)
        pltpu.sync_copy(h3.at[pl.ds(j * CH, CH)], vb)
        pltpu.sync_copy(vb, acc.at[bidx.at[0]], add=True)
        pltpu.sync_copy(onesv, cacc.at[bidx.at[0]], add=True)

    plsc.subcore_barrier()

    @pl.when(jnp.logical_and(s == 0, c == 0))
    def _():
        pltpu.sync_copy(acc, ps0)
        pltpu.sync_copy(cacc, pc0)

    @pl.when(jnp.logical_and(s == 0, c == 1))
    def _():
        pltpu.sync_copy(acc, ps1)
        pltpu.sync_copy(cacc, pc1)


def _make_agg16():
    return pl.kernel(
        _agg16_body,
        out_type=(
            jax.ShapeDtypeStruct((NPAD, 16), _f32),
            jax.ShapeDtypeStruct((NPAD, 16), _f32),
        ),
        mesh=_mesh(),
        compiler_params=pltpu.CompilerParams(use_tc_tiling_on_sc=False),
        scratch_types=(
            pltpu.VMEM_SHARED((NPAD, 16), _f32),
            pltpu.VMEM((GRP, CH), jnp.int32),
            pltpu.VMEM((GRP, CH), jnp.int32),
        ) + tuple(pltpu.VMEM((CH, 16), _f32) for _ in range(8))
        + tuple(pltpu.SemaphoreType.DMA for _ in range(16)),
    )


def _make_agg32():
    return pl.kernel(
        _agg32_body,
        out_type=(
            jax.ShapeDtypeStruct((NPAD, 32), _f32),
            jax.ShapeDtypeStruct((NPAD, 32), _f32),
        ),
        mesh=_mesh(),
        compiler_params=pltpu.CompilerParams(use_tc_tiling_on_sc=False),
        scratch_types=(
            pltpu.VMEM_SHARED((NPAD, 32), _f32),
            pltpu.VMEM((GRP, CH), jnp.int32),
            pltpu.VMEM((GRP, CH), jnp.int32),
            pltpu.VMEM((CH, 32), _f32),
            pltpu.VMEM((CH, 32), _f32),
            pltpu.VMEM((CH, 32), _f32),
            pltpu.VMEM((CH, 32), _f32),
            pltpu.SemaphoreType.DMA,
            pltpu.SemaphoreType.DMA,
            pltpu.SemaphoreType.DMA,
            pltpu.SemaphoreType.DMA,
            pltpu.SemaphoreType.DMA,
            pltpu.SemaphoreType.DMA,
            pltpu.SemaphoreType.DMA,
            pltpu.SemaphoreType.DMA,
        ),
    )


def _make_pool():
    return pl.kernel(
        _pool_body,
        out_type=(
            jax.ShapeDtypeStruct((GPAD, 64), _f32),
            jax.ShapeDtypeStruct((GPAD, 16), _f32),
            jax.ShapeDtypeStruct((GPAD, 64), _f32),
            jax.ShapeDtypeStruct((GPAD, 16), _f32),
        ),
        mesh=_mesh(),
        compiler_params=pltpu.CompilerParams(use_tc_tiling_on_sc=False),
        scratch_types=(
            pltpu.VMEM_SHARED((GPAD, 64), _f32),
            pltpu.VMEM_SHARED((GPAD, 16), _f32),
            pltpu.VMEM((1, CH), jnp.int32),
            pltpu.VMEM((CH, 64), _f32),
            pltpu.VMEM((CH, 16), _f32),
        ),
    )


def _dense1_body(p0, p1, x, wl, wr, b, ha, hb):
    a = p0[...] + p1[...]
    rec = 1.0 / jnp.maximum(a[:, 4:5], 1.0)
    mean4 = a[:, 0:4] * rec
    h = (jnp.dot(mean4, wl[...], preferred_element_type=_f32)
         + jnp.dot(x[...], wr[...], preferred_element_type=_f32) + b[...])
    h = jnp.maximum(h, 0.0)
    ha[...] = h[:, 0:32]
    hb[...] = h[:, 32:64]


def _dense23_body(sa, sb, p0, p1, ha, hb, wla, wlb, wra, wrb, b,
                  *outs, relu, split):
    cnt = p0[:, 4:5] + p1[:, 4:5]
    rec = 1.0 / jnp.maximum(cnt, 1.0)
    h = (jnp.dot(sa[...] * rec, wla[...], preferred_element_type=_f32)
         + jnp.dot(sb[...] * rec, wlb[...], preferred_element_type=_f32)
         + jnp.dot(ha[...], wra[...], preferred_element_type=_f32)
         + jnp.dot(hb[...], wrb[...], preferred_element_type=_f32)
         + b[...])
    if relu:
        h = jnp.maximum(h, 0.0)
    if split:
        outs[0][...] = h[:, 0:32]
        outs[1][...] = h[:, 32:64]
    else:
        outs[0][...] = h


def _head_body(ps0, ps1, pc0, pc1, goal, impw, impb, l1wa, l1wg, l1b,
               bn1g, bn1b, l2w, l2b, bn2g, bn2b, l3w, l3b, bn3g, bn3b,
               loutw, loutb, im_out, label_out):
    ps = ps0[...] + ps1[...]
    pc = pc0[:, 0:1] + pc1[:, 0:1]
    pooled = ps[0:G] / jnp.maximum(pc[0:G], 1.0)
    imp = jnp.dot(pooled, impw[...], preferred_element_type=_f32) + impb[...]
    nrm = jnp.sqrt(jnp.sum(imp * imp, axis=1, keepdims=True))
    im = imp / jnp.maximum(nrm, 1e-12)
    im_out[...] = im

    def bn(t, g, b):
        m = jnp.mean(t, axis=0, keepdims=True)
        d = t - m
        v = jnp.mean(d * d, axis=0, keepdims=True)
        return g * d / jnp.sqrt(v + 1e-5) + b

    t = (jnp.dot(jnp.abs(im), l1wa[...], preferred_element_type=_f32)
         + jnp.dot(goal[...], l1wg[...], preferred_element_type=_f32)
         + l1b[...])
    t = jnp.maximum(bn(t, bn1g[...], bn1b[...]), 0.0)
    t = jnp.dot(t, l2w[...], preferred_element_type=_f32) + l2b[...]
    t = jnp.maximum(bn(t, bn2g[...], bn2b[...]), 0.0)
    t = jnp.dot(t, l3w[...], preferred_element_type=_f32) + l3b[...]
    t = jnp.maximum(bn(t, bn3g[...], bn3b[...]), 0.0)
    label_out[...] = (jnp.dot(t, loutw[...], preferred_element_type=_f32)
                      + loutb[...])


def _row_spec(cols):
    return pl.BlockSpec((NB, cols), lambda i: (i, 0))


def _full_spec(shape):
    nd = len(shape)
    return pl.BlockSpec(shape, lambda i: (0,) * nd)


def _dense1(p0, p1, x, wl, wr, b):
    return pl.pallas_call(
        _dense1_body,
        grid=(NBLK,),
        in_specs=[
            _row_spec(16), _row_spec(16), _row_spec(4),
            _full_spec((4, 64)), _full_spec((4, 64)), _full_spec((1, 64)),
        ],
        out_specs=[_row_spec(32), _row_spec(32)],
        out_shape=[
            jax.ShapeDtypeStruct((N, 32), _f32),
            jax.ShapeDtypeStruct((N, 32), _f32),
        ],
    )(p0, p1, x, wl, wr, b)


def _dense23(sa, sb, p0, p1, ha, hb, wla, wlb, wra, wrb, b, relu, split):
    if split:
        out_specs = [_row_spec(32), _row_spec(32)]
        out_shape = [jax.ShapeDtypeStruct((N, 32), _f32),
                     jax.ShapeDtypeStruct((N, 32), _f32)]
    else:
        out_specs = [_row_spec(64)]
        out_shape = [jax.ShapeDtypeStruct((NPAD, 64), _f32)]
    return pl.pallas_call(
        functools.partial(_dense23_body, relu=relu, split=split),
        grid=(NBLK,),
        in_specs=[
            _row_spec(32), _row_spec(32), _row_spec(16), _row_spec(16),
            _row_spec(32), _row_spec(32),
            _full_spec((32, 64)), _full_spec((32, 64)),
            _full_spec((32, 64)), _full_spec((32, 64)), _full_spec((1, 64)),
        ],
        out_specs=out_specs,
        out_shape=out_shape,
    )(sa, sb, p0, p1, ha, hb, wla, wlb, wra, wrb, b)


def _head(ps0, ps1, pc0, pc1, goal, impw, impb, l1wa, l1wg, l1b,
          bn1g, bn1b, l2w, l2b, bn2g, bn2b, l3w, l3b, bn3g, bn3b,
          loutw, loutb):
    args = (ps0, ps1, pc0, pc1, goal, impw, impb, l1wa, l1wg, l1b,
            bn1g, bn1b, l2w, l2b, bn2g, bn2b, l3w, l3b, bn3g, bn3b,
            loutw, loutb)
    return pl.pallas_call(
        _head_body,
        grid=(1,),
        in_specs=[_full_spec(a.shape) for a in args],
        out_specs=[_full_spec((G, 6)), _full_spec((G, 5))],
        out_shape=[jax.ShapeDtypeStruct((G, 6), _f32),
                   jax.ShapeDtypeStruct((G, 5), _f32)],
    )(*args)


def kernel(x, edge_index, batch, goal, conv1_Wl, conv1_Wr, conv1_b,
           conv2_Wl, conv2_Wr, conv2_b, conv3_Wl, conv3_Wr, conv3_b,
           imp_W, imp_b, l1_W, l1_b, bn1_g, bn1_b, l2_W, l2_b, bn2_g,
           bn2_b, l3_W, l3_b, bn3_g, bn3_b, lout_W, lout_b):
    src = edge_index[0]
    dst = edge_index[1]
    srcp = jnp.concatenate(
        [src, jnp.zeros((EPAD - E,), jnp.int32)]).reshape(NCHUNK, CH)
    dstp = jnp.concatenate(
        [dst, jnp.full((EPAD - E,), N, jnp.int32)]).reshape(NCHUNK, CH)
    batchp = jnp.concatenate(
        [batch.astype(jnp.int32),
         jnp.full((NPAD - N,), G, jnp.int32)]).reshape(NPAD // CH, 1, CH)

    xe = jnp.concatenate(
        [x, jnp.ones((N, 1), _f32), jnp.zeros((N, 11), _f32)], axis=1)
    z16 = jnp.zeros((NPAD, 16), _f32)
    z32 = jnp.zeros((NPAD, 32), _f32)
    zg64 = jnp.zeros((GPAD, 64), _f32)
    zg16 = jnp.zeros((GPAD, 16), _f32)
    ones16 = jnp.zeros((CH, 16), _f32).at[:, 0].set(1.0)

    p0, p1 = _make_agg16()(xe, srcp, dstp, z16)
    h1a, h1b = _dense1(p0, p1, x, conv1_Wl, conv1_Wr,
                       conv1_b.reshape(1, 64))

    s2a, s2b = _make_agg32()(h1a, h1b, srcp, dstp, z32)
    h2a, h2b = _dense23(s2a, s2b, p0, p1, h1a, h1b,
                        conv2_Wl[0:32], conv2_Wl[32:64],
                        conv2_Wr[0:32], conv2_Wr[32:64],
                        conv2_b.reshape(1, 64), relu=True, split=True)

    s3a, s3b = _make_agg32()(h2a, h2b, srcp, dstp, z32)
    (h3,) = _dense23(s3a, s3b, p0, p1, h2a, h2b,
                     conv3_Wl[0:32], conv3_Wl[32:64],
                     conv3_Wr[0:32], conv3_Wr[32:64],
                     conv3_b.reshape(1, 64), relu=False, split=False)

    ps0, pc0, ps1, pc1 = _make_pool()(h3, batchp, ones16, zg64, zg16)

    im, label = _head(
        ps0, ps1, pc0, pc1, goal, imp_W, imp_b.reshape(1, 6),
        l1_W[0:6], l1_W[6:9], l1_b.reshape(1, 512),
        bn1_g.reshape(1, 512), bn1_b.reshape(1, 512),
        l2_W, l2_b.reshape(1, 128), bn2_g.reshape(1, 128),
        bn2_b.reshape(1, 128), l3_W, l3_b.reshape(1, 64),
        bn3_g.reshape(1, 64), bn3_b.reshape(1, 64),
        lout_W, lout_b.reshape(1, 5))
    return (im, label)


# speedup vs baseline: 1.5688x; 1.5688x over previous
import functools

import jax
import jax.numpy as jnp
from jax import lax
from jax.experimental import pallas as pl
from jax.experimental.pallas import tpu as pltpu
from jax.experimental.pallas import tpu_sc as plsc

N = 50000
E = 800000
G = 512
H = 64

NC = 2
NS = 16
NW = NC * NS
CH = 128
CPW = 200
EPAD = NW * CPW * CH
NCHUNK = EPAD // CH
NPAD = 50048
RPT = NPAD // NS
GPAD = 520
NB = 3128
NBLK = NPAD // NB
NDEPTH = 4
GRP = 40
PRE = 2

_f32 = jnp.float32


def _mesh():
    return plsc.VectorSubcoreMesh(core_axis_name="c", subcore_axis_name="s")


def _edge_phase(val_hbm, srcp, dstp, sidx, didx, rbufs, sems, acc,
                base0, ngroups):
    gsems, ssems = sems
    ndepth = len(rbufs)
    pre = ndepth // 2

    @pl.loop(0, ngroups)
    def _groups(g):
        base = base0 + g * GRP
        pltpu.sync_copy(srcp.at[pl.ds(base, GRP)], sidx)
        pltpu.sync_copy(dstp.at[pl.ds(base, GRP)], didx)
        for b in range(pre):
            pltpu.async_copy(val_hbm.at[sidx.at[b]], rbufs[b], gsems[b])

        @pl.loop(0, GRP, step=ndepth)
        def _chunks(k):
            for b in range(ndepth):
                j = k + b
                pltpu.make_async_copy(
                    val_hbm.at[pl.ds(0, CH)], rbufs[b], gsems[b]
                ).wait()
                pltpu.async_copy(rbufs[b], acc.at[didx.at[j]], ssems[b],
                                 add=True)
                b2 = (b + pre) % ndepth

                @pl.when(j >= pre)
                def _():
                    pltpu.make_async_copy(
                        rbufs[b2], acc.at[pl.ds(0, CH)], ssems[b2]
                    ).wait()

                @pl.when(j + pre < GRP)
                def _():
                    pltpu.async_copy(val_hbm.at[sidx.at[j + pre]],
                                     rbufs[b2], gsems[b2])

        for j in range(GRP - pre, GRP):
            b = j % ndepth
            pltpu.make_async_copy(
                rbufs[b], acc.at[pl.ds(0, CH)], ssems[b]
            ).wait()


def _agg16_body(xe, srcp, dstp, z16, p0, p1,
                acc, sidx, didx, rb0, rb1, rb2, rb3, rb4, rb5, rb6, rb7,
                gs0, gs1, gs2, gs3, gs4, gs5, gs6, gs7,
                ss0, ss1, ss2, ss3, ss4, ss5, ss6, ss7):
    c = lax.axis_index("c")
    s = lax.axis_index("s")
    w = s * NC + c
    r0 = s * RPT
    pltpu.sync_copy(z16.at[pl.ds(r0, RPT)], acc.at[pl.ds(r0, RPT)])
    plsc.subcore_barrier()

    _edge_phase(xe, srcp, dstp, sidx, didx,
                (rb0, rb1, rb2, rb3, rb4, rb5, rb6, rb7),
                ((gs0, gs1, gs2, gs3, gs4, gs5, gs6, gs7),
                 (ss0, ss1, ss2, ss3, ss4, ss5, ss6, ss7)),
                acc, w * CPW, CPW // GRP)
    plsc.subcore_barrier()

    @pl.when(c == 0)
    def _():
        pltpu.sync_copy(acc.at[pl.ds(r0, RPT)], p0.at[pl.ds(r0, RPT)])

    @pl.when(c == 1)
    def _():
        pltpu.sync_copy(acc.at[pl.ds(r0, RPT)], p1.at[pl.ds(r0, RPT)])


def _agg32_body(ha, hb, srcp, dstp, z32, sa, sb,
                acc, sidx, didx, rb0, rb1, rb2, rb3,
                gs0, gs1, gs2, gs3, ss0, ss1, ss2, ss3):
    c = lax.axis_index("c")
    s = lax.axis_index("s")
    r0 = s * RPT
    pltpu.sync_copy(z32.at[pl.ds(r0, RPT)], acc.at[pl.ds(r0, RPT)])
    plsc.subcore_barrier()

    rbufs = (rb0, rb1, rb2, rb3)
    sems = ((gs0, gs1, gs2, gs3), (ss0, ss1, ss2, ss3))
    base0 = s * (2 * CPW)
    ngroups = 2 * CPW // GRP

    @pl.when(c == 0)
    def _():
        _edge_phase(ha, srcp, dstp, sidx, didx, rbufs, sems, acc,
                    base0, ngroups)

    @pl.when(c == 1)
    def _():
        _edge_phase(hb, srcp, dstp, sidx, didx, rbufs, sems, acc,
                    base0, ngroups)

    plsc.subcore_barrier()

    @pl.when(c == 0)
    def _():
        pltpu.sync_copy(acc.at[pl.ds(r0, RPT)], sa.at[pl.ds(r0, RPT)])

    @pl.when(c == 1)
    def _():
        pltpu.sync_copy(acc.at[pl.ds(r0, RPT)], sb.at[pl.ds(r0, RPT)])


def _pool_body(h3, batchp, ones16, zg64, zg16, ps0, pc0, ps1, pc1,
               acc, cacc, bidx, vb, onesv):
    c = lax.axis_index("c")
    s = lax.axis_index("s")
    w = s * NC + c

    @pl.when(s == 0)
    def _():
        pltpu.sync_copy(zg64, acc)
        pltpu.sync_copy(zg16, cacc)

    pltpu.sync_copy(ones16, onesv)
    plsc.subcore_barrier()

    nrows = NPAD // CH
    nj = (nrows - w + NW - 1) // NW

    @pl.loop(0, nj)
    def _rows(k):
        j = w + k * NW
        pltpu.sync_copy(batchp.at[j], bidx)
        pltpu.sync_copy(h3.at[pl.ds(j * CH, CH)], vb)
        pltpu.sync_copy(vb, acc.at[bidx.at[0]], add=True)
        pltpu.sync_copy(onesv, cacc.at[bidx.at[0]], add=True)

    plsc.subcore_barrier()

    @pl.when(jnp.logical_and(s == 0, c == 0))
    def _():
        pltpu.sync_copy(acc, ps0)
        pltpu.sync_copy(cacc, pc0)

    @pl.when(jnp.logical_and(s == 0, c == 1))
    def _():
        pltpu.sync_copy(acc, ps1)
        pltpu.sync_copy(cacc, pc1)


def _make_agg16():
    return pl.kernel(
        _agg16_body,
        out_type=(
            jax.ShapeDtypeStruct((NPAD, 16), _f32),
            jax.ShapeDtypeStruct((NPAD, 16), _f32),
        ),
        mesh=_mesh(),
        compiler_params=pltpu.CompilerParams(use_tc_tiling_on_sc=False),
        scratch_types=(
            pltpu.VMEM_SHARED((NPAD, 16), _f32),
            pltpu.VMEM((GRP, CH), jnp.int32),
            pltpu.VMEM((GRP, CH), jnp.int32),
        ) + tuple(pltpu.VMEM((CH, 16), _f32) for _ in range(8))
        + tuple(pltpu.SemaphoreType.DMA for _ in range(16)),
    )


def _make_agg32():
    return pl.kernel(
        _agg32_body,
        out_type=(
            jax.ShapeDtypeStruct((NPAD, 32), _f32),
            jax.ShapeDtypeStruct((NPAD, 32), _f32),
        ),
        mesh=_mesh(),
        compiler_params=pltpu.CompilerParams(use_tc_tiling_on_sc=False),
        scratch_types=(
            pltpu.VMEM_SHARED((NPAD, 32), _f32),
            pltpu.VMEM((GRP, CH), jnp.int32),
            pltpu.VMEM((GRP, CH), jnp.int32),
            pltpu.VMEM((CH, 32), _f32),
            pltpu.VMEM((CH, 32), _f32),
            pltpu.VMEM((CH, 32), _f32),
            pltpu.VMEM((CH, 32), _f32),
            pltpu.SemaphoreType.DMA,
            pltpu.SemaphoreType.DMA,
            pltpu.SemaphoreType.DMA,
            pltpu.SemaphoreType.DMA,
            pltpu.SemaphoreType.DMA,
            pltpu.SemaphoreType.DMA,
            pltpu.SemaphoreType.DMA,
            pltpu.SemaphoreType.DMA,
        ),
    )


def _make_pool():
    return pl.kernel(
        _pool_body,
        out_type=(
            jax.ShapeDtypeStruct((GPAD, 64), _f32),
            jax.ShapeDtypeStruct((GPAD, 16), _f32),
            jax.ShapeDtypeStruct((GPAD, 64), _f32),
            jax.ShapeDtypeStruct((GPAD, 16), _f32),
        ),
        mesh=_mesh(),
        compiler_params=pltpu.CompilerParams(use_tc_tiling_on_sc=False),
        scratch_types=(
            pltpu.VMEM_SHARED((GPAD, 64), _f32),
            pltpu.VMEM_SHARED((GPAD, 16), _f32),
            pltpu.VMEM((1, CH), jnp.int32),
            pltpu.VMEM((CH, 64), _f32),
            pltpu.VMEM((CH, 16), _f32),
        ),
    )


def _dense1_body(p0, p1, x, wl, wr, b, ha, hb):
    a = p0[...] + p1[...]
    rec = 1.0 / jnp.maximum(a[:, 4:5], 1.0)
    mean4 = a[:, 0:4] * rec
    h = (jnp.dot(mean4, wl[...], preferred_element_type=_f32)
         + jnp.dot(x[...], wr[...], preferred_element_type=_f32) + b[...])
    h = jnp.maximum(h, 0.0)
    ha[...] = h[:, 0:32]
    hb[...] = h[:, 32:64]


def _dense23_body(sa, sb, p0, p1, ha, hb, wla, wlb, wra, wrb, b,
                  *outs, relu, split):
    cnt = p0[:, 4:5] + p1[:, 4:5]
    rec = 1.0 / jnp.maximum(cnt, 1.0)
    h = (jnp.dot(sa[...] * rec, wla[...], preferred_element_type=_f32)
         + jnp.dot(sb[...] * rec, wlb[...], preferred_element_type=_f32)
         + jnp.dot(ha[...], wra[...], preferred_element_type=_f32)
         + jnp.dot(hb[...], wrb[...], preferred_element_type=_f32)
         + b[...])
    if relu:
        h = jnp.maximum(h, 0.0)
    if split:
        outs[0][...] = h[:, 0:32]
        outs[1][...] = h[:, 32:64]
    else:
        outs[0][...] = h


def _head_body(ps0, ps1, pc0, pc1, goal, impw, impb, l1wa, l1wg, l1b,
               bn1g, bn1b, l2w, l2b, bn2g, bn2b, l3w, l3b, bn3g, bn3b,
               loutw, loutb, im_out, label_out):
    ps = ps0[...] + ps1[...]
    pc = pc0[:, 0:1] + pc1[:, 0:1]
    pooled = ps[0:G] / jnp.maximum(pc[0:G], 1.0)
    imp = jnp.dot(pooled, impw[...], preferred_element_type=_f32) + impb[...]
    nrm = jnp.sqrt(jnp.sum(imp * imp, axis=1, keepdims=True))
    im = imp / jnp.maximum(nrm, 1e-12)
    im_out[...] = im

    def bn(t, g, b):
        m = jnp.mean(t, axis=0, keepdims=True)
        d = t - m
        v = jnp.mean(d * d, axis=0, keepdims=True)
        return g * d / jnp.sqrt(v + 1e-5) + b

    t = (jnp.dot(jnp.abs(im), l1wa[...], preferred_element_type=_f32)
         + jnp.dot(goal[...], l1wg[...], preferred_element_type=_f32)
         + l1b[...])
    t = jnp.maximum(bn(t, bn1g[...], bn1b[...]), 0.0)
    t = jnp.dot(t, l2w[...], preferred_element_type=_f32) + l2b[...]
    t = jnp.maximum(bn(t, bn2g[...], bn2b[...]), 0.0)
    t = jnp.dot(t, l3w[...], preferred_element_type=_f32) + l3b[...]
    t = jnp.maximum(bn(t, bn3g[...], bn3b[...]), 0.0)
    label_out[...] = (jnp.dot(t, loutw[...], preferred_element_type=_f32)
                      + loutb[...])


def _row_spec(cols):
    return pl.BlockSpec((NB, cols), lambda i: (i, 0))


def _full_spec(shape):
    nd = len(shape)
    return pl.BlockSpec(shape, lambda i: (0,) * nd)


def _dense1(p0, p1, x, wl, wr, b):
    return pl.pallas_call(
        _dense1_body,
        grid=(NBLK,),
        in_specs=[
            _row_spec(16), _row_spec(16), _row_spec(4),
            _full_spec((4, 64)), _full_spec((4, 64)), _full_spec((1, 64)),
        ],
        out_specs=[_row_spec(32), _row_spec(32)],
        out_shape=[
            jax.ShapeDtypeStruct((N, 32), _f32),
            jax.ShapeDtypeStruct((N, 32), _f32),
        ],
    )(p0, p1, x, wl, wr, b)


def _dense23(sa, sb, p0, p1, ha, hb, wla, wlb, wra, wrb, b, relu, split):
    if split:
        out_specs = [_row_spec(32), _row_spec(32)]
        out_shape = [jax.ShapeDtypeStruct((N, 32), _f32),
                     jax.ShapeDtypeStruct((N, 32), _f32)]
    else:
        out_specs = [_row_spec(64)]
        out_shape = [jax.ShapeDtypeStruct((NPAD, 64), _f32)]
    return pl.pallas_call(
        functools.partial(_dense23_body, relu=relu, split=split),
        grid=(NBLK,),
        in_specs=[
            _row_spec(32), _row_spec(32), _row_spec(16), _row_spec(16),
            _row_spec(32), _row_spec(32),
            _full_spec((32, 64)), _full_spec((32, 64)),
            _full_spec((32, 64)), _full_spec((32, 64)), _full_spec((1, 64)),
        ],
        out_specs=out_specs,
        out_shape=out_shape,
    )(sa, sb, p0, p1, ha, hb, wla, wlb, wra, wrb, b)


def _head(ps0, ps1, pc0, pc1, goal, impw, impb, l1wa, l1wg, l1b,
          bn1g, bn1b, l2w, l2b, bn2g, bn2b, l3w, l3b, bn3g, bn3b,
          loutw, loutb):
    args = (ps0, ps1, pc0, pc1, goal, impw, impb, l1wa, l1wg, l1b,
            bn1g, bn1b, l2w, l2b, bn2g, bn2b, l3w, l3b, bn3g, bn3b,
            loutw, loutb)
    return pl.pallas_call(
        _head_body,
        grid=(1,),
        in_specs=[_full_spec(a.shape) for a in args],
        out_specs=[_full_spec((G, 6)), _full_spec((G, 5))],
        out_shape=[jax.ShapeDtypeStruct((G, 6), _f32),
                   jax.ShapeDtypeStruct((G, 5), _f32)],
    )(*args)


def kernel(x, edge_index, batch, goal, conv1_Wl, conv1_Wr, conv1_b,
           conv2_Wl, conv2_Wr, conv2_b, conv3_Wl, conv3_Wr, conv3_b,
           imp_W, imp_b, l1_W, l1_b, bn1_g, bn1_b, l2_W, l2_b, bn2_g,
           bn2_b, l3_W, l3_b, bn3_g, bn3_b, lout_W, lout_b):
    src = edge_index[0]
    dst = edge_index[1]
    perm = jnp.arange(NCHUNK, dtype=jnp.int32).reshape(CPW, NW).T.reshape(-1)
    npade = EPAD - E
    srcp = jnp.concatenate(
        [src, jnp.arange(npade, dtype=jnp.int32)]).reshape(NCHUNK, CH)[perm]
    dstp = jnp.concatenate(
        [dst, N + jnp.arange(npade, dtype=jnp.int32) % (NPAD - N)]
    ).reshape(NCHUNK, CH)[perm]
    batchp = jnp.concatenate(
        [batch.astype(jnp.int32),
         jnp.full((NPAD - N,), G, jnp.int32)]).reshape(NPAD // CH, 1, CH)

    xe = jnp.concatenate(
        [x, jnp.ones((N, 1), _f32), jnp.zeros((N, 11), _f32)], axis=1)
    z16 = jnp.zeros((NPAD, 16), _f32)
    z32 = jnp.zeros((NPAD, 32), _f32)
    zg64 = jnp.zeros((GPAD, 64), _f32)
    zg16 = jnp.zeros((GPAD, 16), _f32)
    ones16 = jnp.zeros((CH, 16), _f32).at[:, 0].set(1.0)

    p0, p1 = _make_agg16()(xe, srcp, dstp, z16)
    h1a, h1b = _dense1(p0, p1, x, conv1_Wl, conv1_Wr,
                       conv1_b.reshape(1, 64))

    s2a, s2b = _make_agg32()(h1a, h1b, srcp, dstp, z32)
    h2a, h2b = _dense23(s2a, s2b, p0, p1, h1a, h1b,
                        conv2_Wl[0:32], conv2_Wl[32:64],
                        conv2_Wr[0:32], conv2_Wr[32:64],
                        conv2_b.reshape(1, 64), relu=True, split=True)

    s3a, s3b = _make_agg32()(h2a, h2b, srcp, dstp, z32)
    (h3,) = _dense23(s3a, s3b, p0, p1, h2a, h2b,
                     conv3_Wl[0:32], conv3_Wl[32:64],
                     conv3_Wr[0:32], conv3_Wr[32:64],
                     conv3_b.reshape(1, 64), relu=False, split=False)

    ps0, pc0, ps1, pc1 = _make_pool()(h3, batchp, ones16, zg64, zg16)

    im, label = _head(
        ps0, ps1, pc0, pc1, goal, imp_W, imp_b.reshape(1, 6),
        l1_W[0:6], l1_W[6:9], l1_b.reshape(1, 512),
        bn1_g.reshape(1, 512), bn1_b.reshape(1, 512),
        l2_W, l2_b.reshape(1, 128), bn2_g.reshape(1, 128),
        bn2_b.reshape(1, 128), l3_W, l3_b.reshape(1, 64),
        bn3_g.reshape(1, 64), bn3_b.reshape(1, 64),
        lout_W, lout_b.reshape(1, 5))
    return (im, label)
